# baseline (speedup 1.0000x reference)
import functools

import jax
import jax.numpy as jnp
from jax import lax
from jax.experimental import pallas as pl
from jax.experimental.pallas import tpu as pltpu
from jax.experimental.pallas import tpu_sc as plsc

B = 64
DIM = 8192
MAX_VAL = 32
NC = 2
NS = 16
NW = NC * NS
ROWS_PER_W = B // NW
E = 512
N_CHUNKS = DIM // E


def _body(x_hbm, g_hbm, u_hbm, w_hbm, out_hbm, gbuf, rowbuf, xbuf, wbuf, ubuf):
    wid = lax.axis_index("s") * NC + lax.axis_index("c")
    pltpu.sync_copy(w_hbm, wbuf)
    pltpu.sync_copy(u_hbm, ubuf)
    iota = lax.iota(jnp.int32, 16)
    iota_hi = iota + 16
    big = jnp.full((16,), 64, jnp.int32)

    for r in range(ROWS_PER_W):
        b = wid * ROWS_PER_W + r

        def chunk_body(ci, _, b=b):
            pltpu.sync_copy(g_hbm.at[b, pl.ds(ci * E * 32, E * 32)], gbuf)

            def group_body(gi, _):
                out = jnp.zeros((16,), jnp.float32)
                for l in range(16):
                    base = (gi * 16 + l) * 32
                    va = gbuf[pl.ds(base, 16)]
                    vb = gbuf[pl.ds(base + 16, 16)]
                    m = jnp.max(jnp.maximum(va, vb))
                    ta = jnp.where(va == m, iota, big)
                    tb = jnp.where(vb == m, iota_hi, big)
                    idx = jnp.min(jnp.minimum(ta, tb))
                    out = jnp.where(iota == l, idx.astype(jnp.float32), out)
                rowbuf[pl.ds(ci * E + gi * 16, 16)] = out
                return 0

            lax.fori_loop(0, E // 16, group_body, 0)
            return 0

        lax.fori_loop(0, N_CHUNKS, chunk_body, 0)

        pltpu.sync_copy(x_hbm.at[b], xbuf)

        def dot_body(j, accv):
            nv = rowbuf[pl.ds(j * 16, 16)]
            xv = xbuf[pl.ds(j * 16, 16)]
            wv = wbuf[pl.ds(j * 16, 16)]
            return accv + (nv - xv) * wv

        accv = lax.fori_loop(0, DIM // 16, dot_body,
                             jnp.zeros((16,), jnp.float32))
        diff = jnp.sum(accv)
        la = jnp.exp(jnp.full((16,), diff))
        ub = plsc.load_gather(ubuf, [jnp.full((16,), b, jnp.int32)])
        accept = la > ub

        def blend_body(j, _):
            nv = rowbuf[pl.ds(j * 16, 16)]
            xv = xbuf[pl.ds(j * 16, 16)]
            rowbuf[pl.ds(j * 16, 16)] = jnp.where(accept, nv, xv)
            return 0

        lax.fori_loop(0, DIM // 16, blend_body, 0)
        pltpu.sync_copy(rowbuf, out_hbm.at[b])


@jax.jit
def kernel(x, g, u, w):
    g2 = g.reshape(B, DIM * MAX_VAL)
    mesh = plsc.VectorSubcoreMesh(core_axis_name="c", subcore_axis_name="s",
                                  num_cores=NC, num_subcores=NS)
    run = pl.kernel(
        _body,
        out_type=jax.ShapeDtypeStruct((B, DIM), jnp.float32),
        mesh=mesh,
        compiler_params=pltpu.CompilerParams(needs_layout_passes=False),
        scratch_types=[
            pltpu.VMEM((E * 32,), jnp.float32),
            pltpu.VMEM((DIM,), jnp.float32),
            pltpu.VMEM((DIM,), jnp.float32),
            pltpu.VMEM((DIM,), jnp.float32),
            pltpu.VMEM((B,), jnp.float32),
        ],
    )
    return run(x, g2, u, w)


# speedup vs baseline: 1.2134x; 1.2134x over previous
import jax
import jax.numpy as jnp
from jax import lax
from jax.experimental import pallas as pl
from jax.experimental.pallas import tpu as pltpu
from jax.experimental.pallas import tpu_sc as plsc

B = 64
DIM = 8192
MAX_VAL = 32
NC = 2
NS = 16
NW = NC * NS
ROWS_PER_W = B // NW
E = 512
CW = E * MAX_VAL
N_CHUNKS = DIM // E
GROUPS = E // 16


def _body(x_hbm, g_hbm, u_hbm, w_hbm, out_hbm,
          gbuf0, gbuf1, rowbuf, xbuf, wbuf, ubuf, sem0, sem1):
    wid = lax.axis_index("s") * NC + lax.axis_index("c")
    pltpu.sync_copy(w_hbm, wbuf)
    pltpu.sync_copy(u_hbm, ubuf)
    iota = lax.iota(jnp.int32, 16)
    pcs = [iota * MAX_VAL + ((iota + k) & (MAX_VAL - 1)) for k in range(MAX_VAL)]

    def compute_chunk(gbuf, ci):
        def group_body(gi, _):
            gslice = gbuf.at[pl.ds(gi * 16 * MAX_VAL, 16 * MAX_VAL)]
            best = plsc.load_gather(gslice, [pcs[0]])
            bpc = pcs[0]
            for k in range(1, MAX_VAL):
                dk = plsc.load_gather(gslice, [pcs[k]])
                take = dk > best
                bpc = jnp.where(take, pcs[k], bpc)
                best = jnp.maximum(dk, best)
            cand = (bpc & (MAX_VAL - 1)).astype(jnp.float32)
            rowbuf[pl.ds(ci * E + gi * 16, 16)] = cand
            return 0

        lax.fori_loop(0, GROUPS, group_body, 0)

    for r in range(ROWS_PER_W):
        b = wid * ROWS_PER_W + r
        pltpu.async_copy(g_hbm.at[b, pl.ds(0, CW)], gbuf0, sem0)
        pltpu.async_copy(g_hbm.at[b, pl.ds(CW, CW)], gbuf1, sem1)

        def pair_body(p, _, b=b):
            c0 = 2 * p

            pltpu.make_async_copy(g_hbm.at[b, pl.ds(0, CW)], gbuf0, sem0).wait()
            compute_chunk(gbuf0, c0)

            @pl.when(c0 + 2 < N_CHUNKS)
            def _():
                pltpu.async_copy(
                    g_hbm.at[b, pl.ds((c0 + 2) * CW, CW)], gbuf0, sem0)

            pltpu.make_async_copy(g_hbm.at[b, pl.ds(0, CW)], gbuf1, sem1).wait()
            compute_chunk(gbuf1, c0 + 1)

            @pl.when(c0 + 3 < N_CHUNKS)
            def _():
                pltpu.async_copy(
                    g_hbm.at[b, pl.ds((c0 + 3) * CW, CW)], gbuf1, sem1)

            return 0

        lax.fori_loop(0, N_CHUNKS // 2, pair_body, 0)

        pltpu.sync_copy(x_hbm.at[b], xbuf)

        def dot_body(j, accv):
            nv = rowbuf[pl.ds(j * 16, 16)]
            xv = xbuf[pl.ds(j * 16, 16)]
            wv = wbuf[pl.ds(j * 16, 16)]
            return accv + (nv - xv) * wv

        accv = lax.fori_loop(0, DIM // 16, dot_body,
                             jnp.zeros((16,), jnp.float32))
        diff = jnp.sum(accv)
        la = jnp.exp(jnp.full((16,), diff))
        ub = plsc.load_gather(ubuf, [jnp.full((16,), b, jnp.int32)])
        accept = la > ub

        def blend_body(j, _):
            nv = rowbuf[pl.ds(j * 16, 16)]
            xv = xbuf[pl.ds(j * 16, 16)]
            rowbuf[pl.ds(j * 16, 16)] = jnp.where(accept, nv, xv)
            return 0

        lax.fori_loop(0, DIM // 16, blend_body, 0)
        pltpu.sync_copy(rowbuf, out_hbm.at[b])


@jax.jit
def kernel(x, g, u, w):
    g2 = g.reshape(B, DIM * MAX_VAL)
    mesh = plsc.VectorSubcoreMesh(core_axis_name="c", subcore_axis_name="s",
                                  num_cores=NC, num_subcores=NS)
    run = pl.kernel(
        _body,
        out_type=jax.ShapeDtypeStruct((B, DIM), jnp.float32),
        mesh=mesh,
        compiler_params=pltpu.CompilerParams(needs_layout_passes=False),
        scratch_types=[
            pltpu.VMEM((CW,), jnp.float32),
            pltpu.VMEM((CW,), jnp.float32),
            pltpu.VMEM((DIM,), jnp.float32),
            pltpu.VMEM((DIM,), jnp.float32),
            pltpu.VMEM((DIM,), jnp.float32),
            pltpu.VMEM((B,), jnp.float32),
            pltpu.SemaphoreType.DMA,
            pltpu.SemaphoreType.DMA,
        ],
    )
    return run(x, g2, u, w)
